# Initial kernel scaffold; baseline (speedup 1.0000x reference)
import functools

import jax
import jax.numpy as jnp
from jax import lax
from jax.experimental import pallas as pl
from jax.experimental.pallas import tpu as pltpu
from jax.experimental.pallas import tpu_sc as plsc

N = 10000
D = 128
E = 320000
N_LAYER = 3

NC = 2
NS = 16
NW = NC * NS
C = 128

N_PAD = 10240
ROWS_PER_TILE = N_PAD // NS
EDGES_PER_TILE = -(-E // (NW * C)) * C
NCHUNK = EDGES_PER_TILE // C
E_PAD = EDGES_PER_TILE * NW


def _sc_layer_body(x_hbm, srcp_hbm, dstp_hbm, zeros2d_hbm, zeros1d_hbm,
                   ones_hbm, part_hbm, cnt_hbm,
                   acc, cntacc, src_t, dst_t, rows, ones_v, gsem):
    cid = lax.axis_index("c")
    sid = lax.axis_index("s")
    wid = cid * NS + sid

    row0 = sid * ROWS_PER_TILE
    pltpu.sync_copy(zeros2d_hbm, acc.at[pl.ds(row0, ROWS_PER_TILE)])
    pltpu.sync_copy(zeros1d_hbm, cntacc.at[pl.ds(row0, ROWS_PER_TILE)])
    pltpu.sync_copy(srcp_hbm.at[wid], src_t)
    pltpu.sync_copy(dstp_hbm.at[wid], dst_t)
    pltpu.sync_copy(ones_hbm, ones_v)
    plsc.subcore_barrier()

    def chunk(j, carry):
        pltpu.async_copy(x_hbm.at[src_t.at[j]], rows, gsem).wait()
        pltpu.sync_copy(rows, acc.at[dst_t.at[j]], add=True)
        pltpu.sync_copy(ones_v, cntacc.at[dst_t.at[j]], add=True)
        return carry

    lax.fori_loop(0, NCHUNK, chunk, 0)
    plsc.subcore_barrier()

    pltpu.sync_copy(acc.at[pl.ds(row0, ROWS_PER_TILE)],
                    part_hbm.at[cid, pl.ds(row0, ROWS_PER_TILE)])
    pltpu.sync_copy(cntacc.at[pl.ds(row0, ROWS_PER_TILE)],
                    cnt_hbm.at[cid, pl.ds(row0, ROWS_PER_TILE)])


@jax.jit
def _sc_layer(x, srcp, dstp, zeros2d, zeros1d, ones):
    mesh = plsc.VectorSubcoreMesh(core_axis_name="c", subcore_axis_name="s",
                                  num_cores=NC, num_subcores=NS)
    f = pl.kernel(
        _sc_layer_body,
        out_type=(jax.ShapeDtypeStruct((NC, N_PAD, D), jnp.float32),
                  jax.ShapeDtypeStruct((NC, N_PAD), jnp.float32)),
        mesh=mesh,
        scratch_types=[
            pltpu.VMEM_SHARED((N_PAD, D), jnp.float32),
            pltpu.VMEM_SHARED((N_PAD,), jnp.float32),
            pltpu.VMEM((NCHUNK, C), jnp.int32),
            pltpu.VMEM((NCHUNK, C), jnp.int32),
            pltpu.VMEM((C, D), jnp.float32),
            pltpu.VMEM((C,), jnp.float32),
            pltpu.SemaphoreType.DMA,
        ],
    )
    return f(x, srcp, dstp, zeros2d, zeros1d, ones)


def _combine_body(p_ref, x_ref, cnt_ref, o_ref):
    total = p_ref[0] + p_ref[1] + x_ref[...]
    cnt = cnt_ref[0] + cnt_ref[1] + 1.0
    o_ref[...] = total / cnt[:, None]


@jax.jit
def _combine(part, x, cnt):
    blk = 400
    grid = N // blk
    return pl.pallas_call(
        _combine_body,
        grid=(grid,),
        in_specs=[
            pl.BlockSpec((NC, blk, D), lambda i: (0, i, 0)),
            pl.BlockSpec((blk, D), lambda i: (i, 0)),
            pl.BlockSpec((NC, blk), lambda i: (0, i)),
        ],
        out_specs=pl.BlockSpec((blk, D), lambda i: (i, 0)),
        out_shape=jax.ShapeDtypeStruct((N, D), jnp.float32),
    )(part, x, cnt)


def kernel(x, edge_index):
    src = edge_index[0]
    dst = edge_index[1]
    pad = E_PAD - E
    srcp = jnp.concatenate([src, jnp.zeros((pad,), jnp.int32)])
    dstp = jnp.concatenate([dst, jnp.full((pad,), N_PAD - 8, jnp.int32)])
    srcp = srcp.reshape(NW, NCHUNK, C)
    dstp = dstp.reshape(NW, NCHUNK, C)
    zeros2d = jnp.zeros((ROWS_PER_TILE, D), jnp.float32)
    zeros1d = jnp.zeros((ROWS_PER_TILE,), jnp.float32)
    ones = jnp.ones((C,), jnp.float32)

    embeds = [x]
    for _ in range(N_LAYER):
        part, cnt = _sc_layer(embeds[-1], srcp, dstp, zeros2d, zeros1d, ones)
        embeds.append(_combine(part, embeds[-1], cnt))
    return tuple(embeds)


# speedup vs baseline: 6.6284x; 6.6284x over previous
import functools

import jax
import jax.numpy as jnp
from jax import lax
from jax.experimental import pallas as pl
from jax.experimental.pallas import tpu as pltpu
from jax.experimental.pallas import tpu_sc as plsc

N = 10000
D = 128
E = 320000
N_LAYER = 3

NC = 2
NS = 16
NW = NC * NS
C = 128

N_PAD = 10240
ROWS_PER_TILE = N_PAD // NS
EDGES_PER_TILE = -(-E // (NW * C)) * C
NCHUNK = EDGES_PER_TILE // C
E_PAD = EDGES_PER_TILE * NW


def _sc_layer_body(x_hbm, srcp_hbm, dstp_hbm, zeros2d_hbm, zeros1d_hbm,
                   ones_hbm, part_hbm, cnt_hbm,
                   acc, cntacc, src_t, dst_t, rows, ones_v, gsem):
    cid = lax.axis_index("c")
    sid = lax.axis_index("s")
    wid = cid * NS + sid

    row0 = sid * ROWS_PER_TILE
    pltpu.sync_copy(zeros2d_hbm, acc.at[pl.ds(row0, ROWS_PER_TILE)])
    pltpu.sync_copy(zeros1d_hbm, cntacc.at[pl.ds(row0, ROWS_PER_TILE)])
    pltpu.sync_copy(srcp_hbm.at[wid], src_t)
    pltpu.sync_copy(dstp_hbm.at[wid], dst_t)
    pltpu.sync_copy(ones_hbm, ones_v)
    plsc.subcore_barrier()

    def chunk(j, carry):
        pltpu.async_copy(x_hbm.at[src_t.at[j]], rows, gsem).wait()
        pltpu.sync_copy(rows, acc.at[dst_t.at[j]], add=True)
        pltpu.sync_copy(ones_v, cntacc.at[dst_t.at[j]], add=True)
        return carry

    lax.fori_loop(0, NCHUNK, chunk, 0)
    plsc.subcore_barrier()

    pltpu.sync_copy(acc.at[pl.ds(row0, ROWS_PER_TILE)],
                    part_hbm.at[cid, pl.ds(row0, ROWS_PER_TILE)])
    pltpu.sync_copy(cntacc.at[pl.ds(row0, ROWS_PER_TILE)],
                    cnt_hbm.at[cid, pl.ds(row0, ROWS_PER_TILE)])


@jax.jit
def _sc_layer(x, srcp, dstp, zeros2d, zeros1d, ones):
    mesh = plsc.VectorSubcoreMesh(core_axis_name="c", subcore_axis_name="s",
                                  num_cores=NC, num_subcores=NS)
    f = pl.kernel(
        _sc_layer_body,
        out_type=(jax.ShapeDtypeStruct((NC, N_PAD, D), jnp.float32),
                  jax.ShapeDtypeStruct((NC, N_PAD), jnp.float32)),
        mesh=mesh,
        scratch_types=[
            pltpu.VMEM_SHARED((N_PAD, D), jnp.float32),
            pltpu.VMEM_SHARED((N_PAD,), jnp.float32),
            pltpu.VMEM((NCHUNK, C), jnp.int32),
            pltpu.VMEM((NCHUNK, C), jnp.int32),
            pltpu.VMEM((C, D), jnp.float32),
            pltpu.VMEM((C,), jnp.float32),
            pltpu.SemaphoreType.DMA,
        ],
    )
    return f(x, srcp, dstp, zeros2d, zeros1d, ones)


def _combine_body(p_ref, x_ref, cnt_ref, o_ref):
    total = p_ref[0] + p_ref[1] + x_ref[...]
    cnt = jnp.sum(cnt_ref[...], axis=1) + 1.0
    o_ref[...] = total / cnt[:, None]


@jax.jit
def _combine(part, x, cnt):
    blk = 400
    grid = N // blk
    return pl.pallas_call(
        _combine_body,
        grid=(grid,),
        in_specs=[
            pl.BlockSpec((NC, blk, D), lambda i: (0, i, 0)),
            pl.BlockSpec((blk, D), lambda i: (i, 0)),
            pl.BlockSpec((blk, NC), lambda i: (i, 0)),
        ],
        out_specs=pl.BlockSpec((blk, D), lambda i: (i, 0)),
        out_shape=jax.ShapeDtypeStruct((N, D), jnp.float32),
    )(part, x, cnt)


def kernel(x, edge_index):
    src = edge_index[0]
    dst = edge_index[1]
    pad = E_PAD - E
    srcp = jnp.concatenate([src, jnp.zeros((pad,), jnp.int32)])
    dstp = jnp.concatenate([dst, jnp.full((pad,), N_PAD - 8, jnp.int32)])
    srcp = srcp.reshape(NW, NCHUNK, C)
    dstp = dstp.reshape(NW, NCHUNK, C)
    zeros2d = jnp.zeros((ROWS_PER_TILE, D), jnp.float32)
    zeros1d = jnp.zeros((ROWS_PER_TILE,), jnp.float32)
    ones = jnp.ones((C,), jnp.float32)

    embeds = [x]
    for _ in range(N_LAYER):
        part, cnt = _sc_layer(embeds[-1], srcp, dstp, zeros2d, zeros1d, ones)
        embeds.append(_combine(part, embeds[-1], cnt.T))
    return tuple(embeds)


# speedup vs baseline: 7.4943x; 1.1306x over previous
import functools

import jax
import jax.numpy as jnp
from jax import lax
from jax.experimental import pallas as pl
from jax.experimental.pallas import tpu as pltpu
from jax.experimental.pallas import tpu_sc as plsc

N = 10000
D = 128
E = 320000
N_LAYER = 3

NC = 2
NS = 16
NW = NC * NS
C = 128

N_PAD = 10240
ROWS_PER_TILE = N_PAD // NS
EDGES_PER_TILE = -(-E // (NW * C)) * C
NCHUNK = EDGES_PER_TILE // C
E_PAD = EDGES_PER_TILE * NW


def _sc_layer_body(x_hbm, srcp_hbm, dstp_hbm, zeros2d_hbm, zeros1d_hbm,
                   ones_hbm, part_hbm, cnt_hbm,
                   acc, cntacc, dst_t, s0, s1, rows0, rows1, ones_v,
                   gsem0, gsem1, ssem0, ssem1):
    cid = lax.axis_index("c")
    sid = lax.axis_index("s")
    wid = cid * NS + sid

    row0 = sid * ROWS_PER_TILE
    pltpu.sync_copy(zeros2d_hbm, acc.at[pl.ds(row0, ROWS_PER_TILE)])
    pltpu.sync_copy(zeros1d_hbm, cntacc.at[pl.ds(row0, ROWS_PER_TILE)])
    pltpu.sync_copy(dstp_hbm.at[wid], dst_t)
    pltpu.sync_copy(ones_hbm, ones_v)
    plsc.subcore_barrier()

    def fire_src(j, buf, sem):
        pltpu.async_copy(srcp_hbm.at[wid, pl.ds(j, 1)], buf, sem)

    def swait(j, buf, sem):
        pltpu.make_async_copy(srcp_hbm.at[wid, pl.ds(j, 1)], buf, sem).wait()

    def fire_gather(buf, idx, sem):
        pltpu.async_copy(x_hbm.at[idx.at[0]], buf, sem)

    def gwait(buf, idx, sem):
        pltpu.make_async_copy(x_hbm.at[idx.at[0]], buf, sem).wait()

    def scat(j, buf):
        pltpu.sync_copy(buf, acc.at[dst_t.at[j]], add=True)
        pltpu.sync_copy(ones_v, cntacc.at[dst_t.at[j]], add=True)

    last = NCHUNK - 1
    fire_src(0, s0, ssem0)
    fire_src(1, s1, ssem1)
    swait(0, s0, ssem0)
    fire_gather(rows0, s0, gsem0)

    def pair(i, carry):
        j0 = 2 * i
        gwait(rows0, s0, gsem0)
        swait(j0 + 1, s1, ssem1)
        fire_gather(rows1, s1, gsem1)
        fire_src(j0 + 2, s0, ssem0)
        scat(j0, rows0)
        gwait(rows1, s1, gsem1)
        swait(j0 + 2, s0, ssem0)
        fire_gather(rows0, s0, gsem0)
        fire_src(jnp.minimum(j0 + 3, last), s1, ssem1)
        scat(j0 + 1, rows1)
        return carry

    lax.fori_loop(0, (NCHUNK - 1) // 2, pair, 0)
    gwait(rows0, s0, gsem0)
    swait(last, s1, ssem1)
    scat(last, rows0)
    plsc.subcore_barrier()

    pltpu.sync_copy(acc.at[pl.ds(row0, ROWS_PER_TILE)],
                    part_hbm.at[cid, pl.ds(row0, ROWS_PER_TILE)])
    pltpu.sync_copy(cntacc.at[pl.ds(row0, ROWS_PER_TILE)],
                    cnt_hbm.at[cid, pl.ds(row0, ROWS_PER_TILE)])


@jax.jit
def _sc_layer(x, srcp, dstp, zeros2d, zeros1d, ones):
    mesh = plsc.VectorSubcoreMesh(core_axis_name="c", subcore_axis_name="s",
                                  num_cores=NC, num_subcores=NS)
    f = pl.kernel(
        _sc_layer_body,
        out_type=(jax.ShapeDtypeStruct((NC, N_PAD, D), jnp.float32),
                  jax.ShapeDtypeStruct((NC, N_PAD), jnp.float32)),
        mesh=mesh,
        scratch_types=[
            pltpu.VMEM_SHARED((N_PAD, D), jnp.float32),
            pltpu.VMEM_SHARED((N_PAD,), jnp.float32),
            pltpu.VMEM((NCHUNK, C), jnp.int32),
            pltpu.VMEM((1, C), jnp.int32),
            pltpu.VMEM((1, C), jnp.int32),
            pltpu.VMEM((C, D), jnp.float32),
            pltpu.VMEM((C, D), jnp.float32),
            pltpu.VMEM((C,), jnp.float32),
            pltpu.SemaphoreType.DMA,
            pltpu.SemaphoreType.DMA,
            pltpu.SemaphoreType.DMA,
            pltpu.SemaphoreType.DMA,
        ],
    )
    return f(x, srcp, dstp, zeros2d, zeros1d, ones)


def _combine_body(p_ref, x_ref, cnt_ref, o_ref):
    total = p_ref[0] + p_ref[1] + x_ref[...]
    cnt = jnp.sum(cnt_ref[...], axis=1) + 1.0
    o_ref[...] = total / cnt[:, None]


@jax.jit
def _combine(part, x, cnt):
    blk = 400
    grid = N // blk
    return pl.pallas_call(
        _combine_body,
        grid=(grid,),
        in_specs=[
            pl.BlockSpec((NC, blk, D), lambda i: (0, i, 0)),
            pl.BlockSpec((blk, D), lambda i: (i, 0)),
            pl.BlockSpec((blk, NC), lambda i: (i, 0)),
        ],
        out_specs=pl.BlockSpec((blk, D), lambda i: (i, 0)),
        out_shape=jax.ShapeDtypeStruct((N, D), jnp.float32),
    )(part, x, cnt)


def kernel(x, edge_index):
    src = edge_index[0]
    dst = edge_index[1]
    pad = E_PAD - E
    srcp = jnp.concatenate([src, jnp.zeros((pad,), jnp.int32)])
    dstp = jnp.concatenate([dst, jnp.full((pad,), N_PAD - 8, jnp.int32)])
    srcp = srcp.reshape(NW, NCHUNK, C)
    dstp = dstp.reshape(NW, NCHUNK, C)
    zeros2d = jnp.zeros((ROWS_PER_TILE, D), jnp.float32)
    zeros1d = jnp.zeros((ROWS_PER_TILE,), jnp.float32)
    ones = jnp.ones((C,), jnp.float32)

    embeds = [x]
    for _ in range(N_LAYER):
        part, cnt = _sc_layer(embeds[-1], srcp, dstp, zeros2d, zeros1d, ones)
        embeds.append(_combine(part, embeds[-1], cnt.T))
    return tuple(embeds)
